# Initial kernel scaffold; baseline (speedup 1.0000x reference)
import jax
import jax.numpy as jnp
from jax.experimental import pallas as pl


def kernel(x, edge_index, W_conv, b_conv, ln_g, ln_b, W_skip, b_skip, W_fc1, b_fc1, W_fc2, b_fc2):
    raise NotImplementedError("write your pallas kernel here")


# speedup vs baseline: 30.4412x; 30.4412x over previous
import functools

import jax
import jax.numpy as jnp
from jax import lax
from jax.experimental import pallas as pl
from jax.experimental.pallas import tpu as pltpu
from jax.experimental.pallas import tpu_sc as plsc

N_NODES = 10000
N_EDGES = 320000
D_IN = 128
H_DIM = 32

NC = 2
NS = 16
NW = NC * NS

EPW = N_EDGES // NW
CW = 80
NCHUNK = EPW // CW

NPAD = 10240
RPT = NPAD // NS


def _deg_body(dst_hbm, out_hbm, idx_v, ones_v, buf_v, acc_sh):
    c = lax.axis_index("c")
    s = lax.axis_index("s")
    wid = s * NC + c

    def _zero(i, _):
        buf_v[pl.ds(i * 16, 16)] = jnp.zeros((16,), jnp.float32)
        return 0

    lax.fori_loop(0, RPT // 16, _zero, 0)
    pltpu.sync_copy(buf_v, acc_sh.at[pl.ds(s * RPT, RPT)])
    for i in range(CW // 16):
        ones_v[pl.ds(i * 16, 16)] = jnp.full((16,), 1.0, jnp.float32)
    pltpu.sync_copy(dst_hbm.at[wid], idx_v)
    plsc.subcore_barrier()

    def _scat(j, _):
        pltpu.sync_copy(ones_v, acc_sh.at[idx_v.at[j]], add=True)
        return 0

    lax.fori_loop(0, NCHUNK, _scat, 0)
    plsc.subcore_barrier()
    pltpu.sync_copy(acc_sh.at[pl.ds(s * RPT, RPT)], buf_v)
    pltpu.sync_copy(buf_v, out_hbm.at[c, pl.ds(s * RPT, RPT)])


def _sc_deg(dst3):
    return pl.kernel(
        _deg_body,
        mesh=plsc.VectorSubcoreMesh(core_axis_name="c", subcore_axis_name="s"),
        out_type=jax.ShapeDtypeStruct((NC, NPAD), jnp.float32),
        scratch_types=[
            pltpu.VMEM((NCHUNK, CW), jnp.int32),
            pltpu.VMEM((CW,), jnp.float32),
            pltpu.VMEM((RPT,), jnp.float32),
            pltpu.VMEM_SHARED((NPAD,), jnp.float32),
        ],
        compiler_params=pltpu.CompilerParams(use_tc_tiling_on_sc=False),
    )(dst3)


def _scat_body(g_hbm, src_hbm, dst_hbm, out_hbm, srcv, dstv, rows, obuf, acc_sh):
    c = lax.axis_index("c")
    s = lax.axis_index("s")
    wid = s * NC + c

    def _zero(i, _):
        obuf[i, pl.ds(0, 16)] = jnp.zeros((16,), jnp.float32)
        obuf[i, pl.ds(16, 16)] = jnp.zeros((16,), jnp.float32)
        return 0

    lax.fori_loop(0, RPT, _zero, 0)
    pltpu.sync_copy(obuf, acc_sh.at[pl.ds(s * RPT, RPT)])
    pltpu.sync_copy(src_hbm.at[wid], srcv)
    pltpu.sync_copy(dst_hbm.at[wid], dstv)
    plsc.subcore_barrier()

    def _edge(j, _):
        pltpu.sync_copy(g_hbm.at[srcv.at[j]], rows)
        pltpu.sync_copy(rows, acc_sh.at[dstv.at[j]], add=True)
        return 0

    lax.fori_loop(0, NCHUNK, _edge, 0)
    plsc.subcore_barrier()
    pltpu.sync_copy(acc_sh.at[pl.ds(s * RPT, RPT)], obuf)
    pltpu.sync_copy(obuf, out_hbm.at[c, pl.ds(s * RPT, RPT)])


def _sc_scatter(g, src3, dst3):
    return pl.kernel(
        _scat_body,
        mesh=plsc.VectorSubcoreMesh(core_axis_name="c", subcore_axis_name="s"),
        out_type=jax.ShapeDtypeStruct((NC, NPAD, H_DIM), jnp.float32),
        scratch_types=[
            pltpu.VMEM((NCHUNK, CW), jnp.int32),
            pltpu.VMEM((NCHUNK, CW), jnp.int32),
            pltpu.VMEM((CW, H_DIM), jnp.float32),
            pltpu.VMEM((RPT, H_DIM), jnp.float32),
            pltpu.VMEM_SHARED((NPAD, H_DIM), jnp.float32),
        ],
        compiler_params=pltpu.CompilerParams(use_tc_tiling_on_sc=False),
    )(g, src3, dst3)


BR = 2000


def _pre_body(x_ref, wc_ref, ws_ref, bs_ref, d0_ref, d1_ref,
              g_ref, dinv_ref, res_ref):
    xb = x_ref[...]
    h = jnp.dot(xb, wc_ref[...], preferred_element_type=jnp.float32)
    deg = d0_ref[...] + d1_ref[...]
    dinv = jnp.where(deg > 0, lax.rsqrt(jnp.maximum(deg, 1e-12)), 0.0)
    g_ref[...] = h * dinv
    dinv_ref[...] = dinv
    res_ref[...] = jnp.dot(xb, ws_ref[...],
                           preferred_element_type=jnp.float32) + bs_ref[...]


def _tc_pre(x, W_conv, W_skip, b_skip, deg0, deg1):
    grid = (N_NODES // BR,)
    row = lambda i: (i, 0)
    fix = lambda i: (0, 0)
    return pl.pallas_call(
        _pre_body,
        grid=grid,
        in_specs=[
            pl.BlockSpec((BR, D_IN), row),
            pl.BlockSpec((D_IN, H_DIM), fix),
            pl.BlockSpec((D_IN, H_DIM), fix),
            pl.BlockSpec((1, H_DIM), fix),
            pl.BlockSpec((BR, 1), row),
            pl.BlockSpec((BR, 1), row),
        ],
        out_specs=[
            pl.BlockSpec((BR, H_DIM), row),
            pl.BlockSpec((BR, 1), row),
            pl.BlockSpec((BR, H_DIM), row),
        ],
        out_shape=[
            jax.ShapeDtypeStruct((N_NODES, H_DIM), jnp.float32),
            jax.ShapeDtypeStruct((N_NODES, 1), jnp.float32),
            jax.ShapeDtypeStruct((N_NODES, H_DIM), jnp.float32),
        ],
    )(x, W_conv, W_skip, b_skip, deg0, deg1)


def _post_body(s0_ref, s1_ref, dinv_ref, bc_ref, lg_ref, lb_ref, res_ref,
               w1_ref, b1_ref, w2_ref, b2_ref, out_ref):
    h = (s0_ref[...] + s1_ref[...]) * dinv_ref[...] + bc_ref[...]
    mu = jnp.mean(h, axis=-1, keepdims=True)
    var = jnp.mean((h - mu) ** 2, axis=-1, keepdims=True)
    hn = (h - mu) / jnp.sqrt(var + 1e-5) * lg_ref[...] + lb_ref[...]
    h2 = jnp.maximum(hn, 0.0) + res_ref[...]
    f1 = jnp.maximum(
        jnp.dot(h2, w1_ref[...], preferred_element_type=jnp.float32)
        + b1_ref[...], 0.0)
    f2 = jnp.dot(f1, w2_ref[...], preferred_element_type=jnp.float32) + b2_ref[...]
    out_ref[...] = jnp.maximum(f2, 0.0) + jnp.log1p(jnp.exp(-jnp.abs(f2)))


def _tc_post(s0, s1, dinv, b_conv, ln_g, ln_b, resid, W_fc1, b_fc1, W_fc2, b_fc2):
    grid = (N_NODES // BR,)
    row = lambda i: (i, 0)
    fix = lambda i: (0, 0)
    return pl.pallas_call(
        _post_body,
        grid=grid,
        in_specs=[
            pl.BlockSpec((BR, H_DIM), row),
            pl.BlockSpec((BR, H_DIM), row),
            pl.BlockSpec((BR, 1), row),
            pl.BlockSpec((1, H_DIM), fix),
            pl.BlockSpec((1, H_DIM), fix),
            pl.BlockSpec((1, H_DIM), fix),
            pl.BlockSpec((BR, H_DIM), row),
            pl.BlockSpec((H_DIM, H_DIM // 2), fix),
            pl.BlockSpec((1, H_DIM // 2), fix),
            pl.BlockSpec((H_DIM // 2, 1), fix),
            pl.BlockSpec((1, 1), fix),
        ],
        out_specs=pl.BlockSpec((BR, 1), row),
        out_shape=jax.ShapeDtypeStruct((N_NODES, 1), jnp.float32),
    )(s0, s1, dinv, b_conv, ln_g, ln_b, resid, W_fc1, b_fc1, W_fc2, b_fc2)


def kernel(x, edge_index, W_conv, b_conv, ln_g, ln_b, W_skip, b_skip,
           W_fc1, b_fc1, W_fc2, b_fc2):
    src3 = edge_index[0].reshape(NW, NCHUNK, CW)
    dst3 = edge_index[1].reshape(NW, NCHUNK, CW)

    deg2 = _sc_deg(dst3)
    g, dinv, resid = _tc_pre(
        x, W_conv, W_skip, b_skip.reshape(1, H_DIM),
        deg2[0, :N_NODES, None], deg2[1, :N_NODES, None])
    parts = _sc_scatter(g, src3, dst3)
    out = _tc_post(
        parts[0, :N_NODES], parts[1, :N_NODES], dinv,
        b_conv.reshape(1, H_DIM), ln_g.reshape(1, H_DIM),
        ln_b.reshape(1, H_DIM), resid,
        W_fc1, b_fc1.reshape(1, H_DIM // 2), W_fc2, b_fc2.reshape(1, 1))
    return out[:, 0]


# speedup vs baseline: 49.2269x; 1.6171x over previous
import functools

import jax
import jax.numpy as jnp
from jax import lax
from jax.experimental import pallas as pl
from jax.experimental.pallas import tpu as pltpu
from jax.experimental.pallas import tpu_sc as plsc

N_NODES = 10000
N_EDGES = 320000
D_IN = 128
H_DIM = 32

NC = 2
NS = 16
NW = NC * NS

EPW = N_EDGES // NW
CW = 80
NCHUNK = EPW // CW

NPAD = 10240
RPT = NPAD // NS

NB = 5
NGRP = NCHUNK // NB


def _deg_body(dst_hbm, out_hbm, idx_v, ones_v, buf_v, acc_sh, *sems):
    c = lax.axis_index("c")
    s = lax.axis_index("s")
    wid = s * NC + c

    def _zero(i, _):
        buf_v[pl.ds(i * 16, 16)] = jnp.zeros((16,), jnp.float32)
        return 0

    lax.fori_loop(0, RPT // 16, _zero, 0)
    pltpu.sync_copy(buf_v, acc_sh.at[pl.ds(s * RPT, RPT)])
    for i in range(CW // 16):
        ones_v[pl.ds(i * 16, 16)] = jnp.full((16,), 1.0, jnp.float32)
    pltpu.sync_copy(dst_hbm.at[wid], idx_v)
    plsc.subcore_barrier()

    for b in range(NB):
        pltpu.async_copy(ones_v, acc_sh.at[idx_v.at[b]], sems[b], add=True)

    def _grp(i, _):
        for b in range(NB):
            j = (i + 1) * NB + b
            pltpu.async_copy(ones_v, acc_sh.at[idx_v.at[j]], sems[b], add=True)
            pltpu.make_async_copy(ones_v, acc_sh.at[idx_v.at[j]], sems[b]).wait()
        return 0

    lax.fori_loop(0, NGRP - 1, _grp, 0)
    for b in range(NB):
        pltpu.make_async_copy(ones_v, acc_sh.at[idx_v.at[b]], sems[b]).wait()
    plsc.subcore_barrier()
    pltpu.sync_copy(acc_sh.at[pl.ds(s * RPT, RPT)], buf_v)
    pltpu.sync_copy(buf_v, out_hbm.at[c, pl.ds(s * RPT, RPT)])


def _sc_deg(dst3):
    return pl.kernel(
        _deg_body,
        mesh=plsc.VectorSubcoreMesh(core_axis_name="c", subcore_axis_name="s"),
        out_type=jax.ShapeDtypeStruct((NC, NPAD), jnp.float32),
        scratch_types=[
            pltpu.VMEM((NCHUNK, CW), jnp.int32),
            pltpu.VMEM((CW,), jnp.float32),
            pltpu.VMEM((RPT,), jnp.float32),
            pltpu.VMEM_SHARED((NPAD,), jnp.float32),
        ] + [pltpu.SemaphoreType.DMA] * NB,
        compiler_params=pltpu.CompilerParams(use_tc_tiling_on_sc=False),
    )(dst3)


def _scat_body(g_hbm, src_hbm, dst_hbm, out_hbm, srcv, dstv, rows, obuf, acc_sh, *sems):
    c = lax.axis_index("c")
    s = lax.axis_index("s")
    wid = s * NC + c
    gsem = sems[:NB]
    ssem = sems[NB:]

    def _zero(i, _):
        obuf[i, pl.ds(0, 16)] = jnp.zeros((16,), jnp.float32)
        obuf[i, pl.ds(16, 16)] = jnp.zeros((16,), jnp.float32)
        return 0

    lax.fori_loop(0, RPT, _zero, 0)
    pltpu.sync_copy(obuf, acc_sh.at[pl.ds(s * RPT, RPT)])
    pltpu.sync_copy(src_hbm.at[wid], srcv)
    pltpu.sync_copy(dst_hbm.at[wid], dstv)
    plsc.subcore_barrier()

    for b in range(NB):
        pltpu.async_copy(g_hbm.at[srcv.at[b]], rows.at[b], gsem[b])

    def _grp(i, _):
        for b in range(NB):
            j = i * NB + b
            pltpu.make_async_copy(g_hbm.at[srcv.at[j]], rows.at[b], gsem[b]).wait()
            pltpu.async_copy(rows.at[b], acc_sh.at[dstv.at[j]], ssem[b], add=True)
        for b in range(NB):
            j = i * NB + b
            pltpu.make_async_copy(rows.at[b], acc_sh.at[dstv.at[j]], ssem[b]).wait()
            pltpu.async_copy(g_hbm.at[srcv.at[j + NB]], rows.at[b], gsem[b])
        return 0

    lax.fori_loop(0, NGRP - 1, _grp, 0)
    for b in range(NB):
        j = (NGRP - 1) * NB + b
        pltpu.make_async_copy(g_hbm.at[srcv.at[j]], rows.at[b], gsem[b]).wait()
        pltpu.async_copy(rows.at[b], acc_sh.at[dstv.at[j]], ssem[b], add=True)
    for b in range(NB):
        j = (NGRP - 1) * NB + b
        pltpu.make_async_copy(rows.at[b], acc_sh.at[dstv.at[j]], ssem[b]).wait()
    plsc.subcore_barrier()
    pltpu.sync_copy(acc_sh.at[pl.ds(s * RPT, RPT)], obuf)
    pltpu.sync_copy(obuf, out_hbm.at[c, pl.ds(s * RPT, RPT)])


def _sc_scatter(g, src3, dst3):
    return pl.kernel(
        _scat_body,
        mesh=plsc.VectorSubcoreMesh(core_axis_name="c", subcore_axis_name="s"),
        out_type=jax.ShapeDtypeStruct((NC, NPAD, H_DIM), jnp.float32),
        scratch_types=[
            pltpu.VMEM((NCHUNK, CW), jnp.int32),
            pltpu.VMEM((NCHUNK, CW), jnp.int32),
            pltpu.VMEM((NB, CW, H_DIM), jnp.float32),
            pltpu.VMEM((RPT, H_DIM), jnp.float32),
            pltpu.VMEM_SHARED((NPAD, H_DIM), jnp.float32),
        ] + [pltpu.SemaphoreType.DMA] * (2 * NB),
        compiler_params=pltpu.CompilerParams(use_tc_tiling_on_sc=False),
    )(g, src3, dst3)


BR = 2000


def _pre_body(x_ref, wc_ref, ws_ref, bs_ref, d0_ref, d1_ref,
              g_ref, dinv_ref, res_ref):
    xb = x_ref[...]
    h = jnp.dot(xb, wc_ref[...], preferred_element_type=jnp.float32)
    deg = d0_ref[...] + d1_ref[...]
    dinv = jnp.where(deg > 0, lax.rsqrt(jnp.maximum(deg, 1e-12)), 0.0)
    g_ref[...] = h * dinv
    dinv_ref[...] = dinv
    res_ref[...] = jnp.dot(xb, ws_ref[...],
                           preferred_element_type=jnp.float32) + bs_ref[...]


def _tc_pre(x, W_conv, W_skip, b_skip, deg0, deg1):
    grid = (N_NODES // BR,)
    row = lambda i: (i, 0)
    fix = lambda i: (0, 0)
    return pl.pallas_call(
        _pre_body,
        grid=grid,
        in_specs=[
            pl.BlockSpec((BR, D_IN), row),
            pl.BlockSpec((D_IN, H_DIM), fix),
            pl.BlockSpec((D_IN, H_DIM), fix),
            pl.BlockSpec((1, H_DIM), fix),
            pl.BlockSpec((BR, 1), row),
            pl.BlockSpec((BR, 1), row),
        ],
        out_specs=[
            pl.BlockSpec((BR, H_DIM), row),
            pl.BlockSpec((BR, 1), row),
            pl.BlockSpec((BR, H_DIM), row),
        ],
        out_shape=[
            jax.ShapeDtypeStruct((N_NODES, H_DIM), jnp.float32),
            jax.ShapeDtypeStruct((N_NODES, 1), jnp.float32),
            jax.ShapeDtypeStruct((N_NODES, H_DIM), jnp.float32),
        ],
    )(x, W_conv, W_skip, b_skip, deg0, deg1)


def _post_body(s0_ref, s1_ref, dinv_ref, bc_ref, lg_ref, lb_ref, res_ref,
               w1_ref, b1_ref, w2_ref, b2_ref, out_ref):
    h = (s0_ref[...] + s1_ref[...]) * dinv_ref[...] + bc_ref[...]
    mu = jnp.mean(h, axis=-1, keepdims=True)
    var = jnp.mean((h - mu) ** 2, axis=-1, keepdims=True)
    hn = (h - mu) / jnp.sqrt(var + 1e-5) * lg_ref[...] + lb_ref[...]
    h2 = jnp.maximum(hn, 0.0) + res_ref[...]
    f1 = jnp.maximum(
        jnp.dot(h2, w1_ref[...], preferred_element_type=jnp.float32)
        + b1_ref[...], 0.0)
    f2 = jnp.dot(f1, w2_ref[...], preferred_element_type=jnp.float32) + b2_ref[...]
    out_ref[...] = jnp.maximum(f2, 0.0) + jnp.log1p(jnp.exp(-jnp.abs(f2)))


def _tc_post(s0, s1, dinv, b_conv, ln_g, ln_b, resid, W_fc1, b_fc1, W_fc2, b_fc2):
    grid = (N_NODES // BR,)
    row = lambda i: (i, 0)
    fix = lambda i: (0, 0)
    return pl.pallas_call(
        _post_body,
        grid=grid,
        in_specs=[
            pl.BlockSpec((BR, H_DIM), row),
            pl.BlockSpec((BR, H_DIM), row),
            pl.BlockSpec((BR, 1), row),
            pl.BlockSpec((1, H_DIM), fix),
            pl.BlockSpec((1, H_DIM), fix),
            pl.BlockSpec((1, H_DIM), fix),
            pl.BlockSpec((BR, H_DIM), row),
            pl.BlockSpec((H_DIM, H_DIM // 2), fix),
            pl.BlockSpec((1, H_DIM // 2), fix),
            pl.BlockSpec((H_DIM // 2, 1), fix),
            pl.BlockSpec((1, 1), fix),
        ],
        out_specs=pl.BlockSpec((BR, 1), row),
        out_shape=jax.ShapeDtypeStruct((N_NODES, 1), jnp.float32),
    )(s0, s1, dinv, b_conv, ln_g, ln_b, resid, W_fc1, b_fc1, W_fc2, b_fc2)


def kernel(x, edge_index, W_conv, b_conv, ln_g, ln_b, W_skip, b_skip,
           W_fc1, b_fc1, W_fc2, b_fc2):
    src3 = edge_index[0].reshape(NW, NCHUNK, CW)
    dst3 = edge_index[1].reshape(NW, NCHUNK, CW)

    deg2 = _sc_deg(dst3)
    g, dinv, resid = _tc_pre(
        x, W_conv, W_skip, b_skip.reshape(1, H_DIM),
        deg2[0, :N_NODES, None], deg2[1, :N_NODES, None])
    parts = _sc_scatter(g, src3, dst3)
    out = _tc_post(
        parts[0, :N_NODES], parts[1, :N_NODES], dinv,
        b_conv.reshape(1, H_DIM), ln_g.reshape(1, H_DIM),
        ln_b.reshape(1, H_DIM), resid,
        W_fc1, b_fc1.reshape(1, H_DIM // 2), W_fc2, b_fc2.reshape(1, 1))
    return out[:, 0]


# speedup vs baseline: 57.3025x; 1.1640x over previous
import functools

import jax
import jax.numpy as jnp
from jax import lax
from jax.experimental import pallas as pl
from jax.experimental.pallas import tpu as pltpu
from jax.experimental.pallas import tpu_sc as plsc

N_NODES = 10000
N_EDGES = 320000
D_IN = 128
H_DIM = 32

NC = 2
NS = 16
NW = NC * NS

EPW = N_EDGES // NW
CW = 80
NCHUNK = EPW // CW

NPAD = 10240
RPT = NPAD // NS

NB = 5
NGRP = NCHUNK // NB


def _deg_body(er_hbm, out_hbm, idx_v, ones_v, buf_v, acc_sh, *sems):
    c = lax.axis_index("c")
    s = lax.axis_index("s")
    wid = s * NC + c

    def _zero(i, _):
        buf_v[pl.ds(i * 16, 16)] = jnp.zeros((16,), jnp.float32)
        return 0

    lax.fori_loop(0, RPT // 16, _zero, 0)
    pltpu.sync_copy(buf_v, acc_sh.at[pl.ds(s * RPT, RPT)])
    for i in range(CW // 16):
        ones_v[pl.ds(i * 16, 16)] = jnp.full((16,), 1.0, jnp.float32)
    pltpu.sync_copy(er_hbm.at[1, wid], idx_v)
    plsc.subcore_barrier()

    for b in range(NB):
        pltpu.async_copy(ones_v, acc_sh.at[idx_v.at[b]], sems[b], add=True)

    def _grp(i, _):
        for b in range(NB):
            j = (i + 1) * NB + b
            pltpu.async_copy(ones_v, acc_sh.at[idx_v.at[j]], sems[b], add=True)
            pltpu.make_async_copy(ones_v, acc_sh.at[idx_v.at[j]], sems[b]).wait()
        return 0

    lax.fori_loop(0, NGRP - 1, _grp, 0)
    for b in range(NB):
        pltpu.make_async_copy(ones_v, acc_sh.at[idx_v.at[b]], sems[b]).wait()
    plsc.subcore_barrier()
    pltpu.sync_copy(acc_sh.at[pl.ds(s * RPT, RPT)], buf_v)
    pltpu.sync_copy(buf_v, out_hbm.at[c, pl.ds(s * RPT, RPT)])


def _sc_deg(er):
    return pl.kernel(
        _deg_body,
        mesh=plsc.VectorSubcoreMesh(core_axis_name="c", subcore_axis_name="s"),
        out_type=jax.ShapeDtypeStruct((NC, NPAD), jnp.float32),
        scratch_types=[
            pltpu.VMEM((NCHUNK, CW), jnp.int32),
            pltpu.VMEM((CW,), jnp.float32),
            pltpu.VMEM((RPT,), jnp.float32),
            pltpu.VMEM_SHARED((NPAD,), jnp.float32),
        ] + [pltpu.SemaphoreType.DMA] * NB,
        compiler_params=pltpu.CompilerParams(use_tc_tiling_on_sc=False),
    )(er)


def _scat_body(g_hbm, er_hbm, out_hbm, srcv, dstv, rows, obuf, acc_sh, *sems):
    c = lax.axis_index("c")
    s = lax.axis_index("s")
    wid = s * NC + c
    gsem = sems[:NB]
    ssem = sems[NB:]

    def _zero(i, _):
        obuf[i, pl.ds(0, 16)] = jnp.zeros((16,), jnp.float32)
        obuf[i, pl.ds(16, 16)] = jnp.zeros((16,), jnp.float32)
        return 0

    lax.fori_loop(0, RPT, _zero, 0)
    pltpu.sync_copy(obuf, acc_sh.at[pl.ds(s * RPT, RPT)])
    pltpu.sync_copy(er_hbm.at[0, wid], srcv)
    pltpu.sync_copy(er_hbm.at[1, wid], dstv)
    plsc.subcore_barrier()

    for b in range(NB):
        pltpu.async_copy(g_hbm.at[srcv.at[b]], rows.at[b], gsem[b])

    def _grp(i, _):
        for b in range(NB):
            j = i * NB + b
            pltpu.make_async_copy(g_hbm.at[srcv.at[j]], rows.at[b], gsem[b]).wait()
            pltpu.async_copy(rows.at[b], acc_sh.at[dstv.at[j]], ssem[b], add=True)
        for b in range(NB):
            j = i * NB + b
            pltpu.make_async_copy(rows.at[b], acc_sh.at[dstv.at[j]], ssem[b]).wait()
            pltpu.async_copy(g_hbm.at[srcv.at[j + NB]], rows.at[b], gsem[b])
        return 0

    lax.fori_loop(0, NGRP - 1, _grp, 0)
    for b in range(NB):
        j = (NGRP - 1) * NB + b
        pltpu.make_async_copy(g_hbm.at[srcv.at[j]], rows.at[b], gsem[b]).wait()
        pltpu.async_copy(rows.at[b], acc_sh.at[dstv.at[j]], ssem[b], add=True)
    for b in range(NB):
        j = (NGRP - 1) * NB + b
        pltpu.make_async_copy(rows.at[b], acc_sh.at[dstv.at[j]], ssem[b]).wait()
    plsc.subcore_barrier()
    pltpu.sync_copy(acc_sh.at[pl.ds(s * RPT, RPT)], obuf)
    pltpu.sync_copy(obuf, out_hbm.at[c, pl.ds(s * RPT, RPT)])


def _sc_scatter(g, er):
    return pl.kernel(
        _scat_body,
        mesh=plsc.VectorSubcoreMesh(core_axis_name="c", subcore_axis_name="s"),
        out_type=jax.ShapeDtypeStruct((NC, NPAD, H_DIM), jnp.float32),
        scratch_types=[
            pltpu.VMEM((NCHUNK, CW), jnp.int32),
            pltpu.VMEM((NCHUNK, CW), jnp.int32),
            pltpu.VMEM((NB, CW, H_DIM), jnp.float32),
            pltpu.VMEM((RPT, H_DIM), jnp.float32),
            pltpu.VMEM_SHARED((NPAD, H_DIM), jnp.float32),
        ] + [pltpu.SemaphoreType.DMA] * (2 * NB),
        compiler_params=pltpu.CompilerParams(use_tc_tiling_on_sc=False),
    )(g, er)


BR = 2000


def _pre_body(x_ref, wc_ref, ws_ref, bs_ref, d0_ref, d1_ref,
              g_ref, dinv_ref, res_ref):
    xb = x_ref[...]
    h = jnp.dot(xb, wc_ref[...], preferred_element_type=jnp.float32)
    deg = d0_ref[...] + d1_ref[...]
    dinv = jnp.where(deg > 0, lax.rsqrt(jnp.maximum(deg, 1e-12)), 0.0)
    g_ref[...] = h * dinv
    dinv_ref[...] = dinv
    res_ref[...] = jnp.dot(xb, ws_ref[...],
                           preferred_element_type=jnp.float32) + bs_ref[...]


def _tc_pre(x, W_conv, W_skip, b_skip, deg0, deg1):
    grid = (N_NODES // BR,)
    row = lambda i: (i, 0)
    fix = lambda i: (0, 0)
    return pl.pallas_call(
        _pre_body,
        grid=grid,
        in_specs=[
            pl.BlockSpec((BR, D_IN), row),
            pl.BlockSpec((D_IN, H_DIM), fix),
            pl.BlockSpec((D_IN, H_DIM), fix),
            pl.BlockSpec((1, H_DIM), fix),
            pl.BlockSpec((BR, 1), row),
            pl.BlockSpec((BR, 1), row),
        ],
        out_specs=[
            pl.BlockSpec((BR, H_DIM), row),
            pl.BlockSpec((BR, 1), row),
            pl.BlockSpec((BR, H_DIM), row),
        ],
        out_shape=[
            jax.ShapeDtypeStruct((N_NODES, H_DIM), jnp.float32),
            jax.ShapeDtypeStruct((N_NODES, 1), jnp.float32),
            jax.ShapeDtypeStruct((N_NODES, H_DIM), jnp.float32),
        ],
    )(x, W_conv, W_skip, b_skip, deg0, deg1)


def _post_body(s0_ref, s1_ref, dinv_ref, bc_ref, lg_ref, lb_ref, res_ref,
               w1_ref, b1_ref, w2_ref, b2_ref, out_ref):
    h = (s0_ref[0] + s1_ref[0]) * dinv_ref[...] + bc_ref[...]
    mu = jnp.mean(h, axis=-1, keepdims=True)
    var = jnp.mean((h - mu) ** 2, axis=-1, keepdims=True)
    hn = (h - mu) / jnp.sqrt(var + 1e-5) * lg_ref[...] + lb_ref[...]
    h2 = jnp.maximum(hn, 0.0) + res_ref[...]
    f1 = jnp.maximum(
        jnp.dot(h2, w1_ref[...], preferred_element_type=jnp.float32)
        + b1_ref[...], 0.0)
    f2 = jnp.dot(f1, w2_ref[...], preferred_element_type=jnp.float32) + b2_ref[...]
    out_ref[...] = jnp.maximum(f2, 0.0) + jnp.log1p(jnp.exp(-jnp.abs(f2)))


def _tc_post(parts, dinv, b_conv, ln_g, ln_b, resid, W_fc1, b_fc1, W_fc2, b_fc2):
    grid = (N_NODES // BR,)
    row = lambda i: (i, 0)
    fix = lambda i: (0, 0)
    return pl.pallas_call(
        _post_body,
        grid=grid,
        in_specs=[
            pl.BlockSpec((1, BR, H_DIM), lambda i: (0, i, 0)),
            pl.BlockSpec((1, BR, H_DIM), lambda i: (1, i, 0)),
            pl.BlockSpec((BR, 1), row),
            pl.BlockSpec((1, H_DIM), fix),
            pl.BlockSpec((1, H_DIM), fix),
            pl.BlockSpec((1, H_DIM), fix),
            pl.BlockSpec((BR, H_DIM), row),
            pl.BlockSpec((H_DIM, H_DIM // 2), fix),
            pl.BlockSpec((1, H_DIM // 2), fix),
            pl.BlockSpec((H_DIM // 2, 1), fix),
            pl.BlockSpec((1, 1), fix),
        ],
        out_specs=pl.BlockSpec((BR, 1), row),
        out_shape=jax.ShapeDtypeStruct((N_NODES, 1), jnp.float32),
    )(parts, parts, dinv, b_conv, ln_g, ln_b, resid, W_fc1, b_fc1, W_fc2, b_fc2)


def kernel(x, edge_index, W_conv, b_conv, ln_g, ln_b, W_skip, b_skip,
           W_fc1, b_fc1, W_fc2, b_fc2):
    er = edge_index.reshape(2, NW, NCHUNK, CW)

    deg2 = _sc_deg(er)
    g, dinv, resid = _tc_pre(
        x, W_conv, W_skip, b_skip.reshape(1, H_DIM),
        deg2[0, :N_NODES, None], deg2[1, :N_NODES, None])
    parts = _sc_scatter(g, er)
    out = _tc_post(
        parts, dinv,
        b_conv.reshape(1, H_DIM), ln_g.reshape(1, H_DIM),
        ln_b.reshape(1, H_DIM), resid,
        W_fc1, b_fc1.reshape(1, H_DIM // 2), W_fc2, b_fc2.reshape(1, 1))
    return out[:, 0]
